# baseline (speedup 1.0000x reference)
import jax
import jax.numpy as jnp
from jax import lax
from jax.experimental import pallas as pl
from jax.experimental.pallas import tpu as pltpu
from jax.experimental.pallas import tpu_sc as plsc

NODE_NUM = 50000
DIM = 128
K1 = 3
M = 2
NPAIR = 2 * K1
PWD = NPAIR * NPAIR
BATCH = 4096

_NC = 2
_NS = 16
_NW = _NC * _NS
_PER_W = (2 * BATCH) // _NW
_CH = 128
_LANES = 16
_NTASK = K1 * (_PER_W // _CH)


def _vgather(x, idx):
    return lax.gather(
        x, idx[:, None],
        lax.GatherDimensionNumbers(offset_dims=(), collapsed_slice_dims=(0,),
                                   start_index_map=(0,)),
        (1,), mode=lax.GatherScatterMode.PROMISE_IN_BOUNDS)


def _sc_fused_gather(ids_hbm, rp_hbm, lam_hbm, out_hbm,
                     idx_v, lam_v, idx0, idx1, b0, b1, fb,
                     gsem_a, gsem_b, osem):
    wid = lax.axis_index("s") * _NC + lax.axis_index("c")
    base = wid * _PER_W
    half = base // BATCH
    brow = base - half * BATCH

    pltpu.sync_copy(ids_hbm.at[pl.ds(base, _PER_W)], idx_v)

    pltpu.sync_copy(lam_hbm, lam_v.at[pl.ds(0, 6)])
    lane = lax.iota(jnp.int32, 16)
    f = lax.rem(lane, 6)
    lv = lam_v[...]
    a = _vgather(lv, f)
    b = _vgather(lv, f ^ 1)
    mx = jnp.maximum(a, b)
    ea = jnp.exp(a - mx)
    eb = jnp.exp(b - mx)
    w = ea / (ea + eb)
    wks = []
    for k in range(K1):
        w0 = _vgather(w, jnp.full((16,), 2 * k, jnp.int32))
        w1 = _vgather(w, jnp.full((16,), 2 * k + 1, jnp.int32))
        wks.append((w0, w1))

    tasks = [(k, c0) for k in range(K1) for c0 in range(0, _PER_W, _CH)]
    gsems = [gsem_a, gsem_b]

    def compute_idx(t, p):
        k, c0 = tasks[t]
        for tt in range(_CH // _LANES):
            sls = pl.ds(c0 + tt * _LANES, _LANES)
            sld = pl.ds(tt * _LANES, _LANES)
            v = idx_v[sls]
            idx0[p, sld] = v + (k * NODE_NUM)
            idx1[p, sld] = v + ((K1 + k) * NODE_NUM)

    def fire(t, p):
        h0 = pltpu.async_copy(rp_hbm.at[idx0.at[p]], b0.at[p], gsems[p])
        h1 = pltpu.async_copy(rp_hbm.at[idx1.at[p]], b1.at[p], gsems[p])
        return h0, h1

    handles = {}
    compute_idx(0, 0)
    handles[0] = fire(0, 0)
    owrite = None
    for t in range(_NTASK):
        p = t % 2
        if t + 1 < _NTASK:
            compute_idx(t + 1, 1 - p)
            handles[t + 1] = fire(t + 1, 1 - p)
        h0, h1 = handles.pop(t)
        h0.wait()
        h1.wait()
        k, c0 = tasks[t]
        w0, w1 = wks[k]

        def fuse_row(c, _, p=p, w0=w0, w1=w1):
            for l in range(DIM // _LANES):
                sl = pl.ds(l * _LANES, _LANES)
                fb[p, c, sl] = b0[p, c, sl] * w0 + b1[p, c, sl] * w1
            return 0

        lax.fori_loop(0, _CH, fuse_row, 0, unroll=2)
        if owrite is not None:
            owrite.wait()
        owrite = pltpu.async_copy(
            fb.at[p], out_hbm.at[half * K1 + k, pl.ds(brow + c0, _CH)], osem)
    owrite.wait()


_sc_gather_call = pl.kernel(
    _sc_fused_gather,
    out_type=jax.ShapeDtypeStruct((NPAIR, BATCH, DIM), jnp.float32),
    mesh=plsc.VectorSubcoreMesh(core_axis_name="c", subcore_axis_name="s"),
    scratch_types=[
        pltpu.VMEM((_PER_W,), jnp.int32),
        pltpu.VMEM((_LANES,), jnp.float32),
        pltpu.VMEM((2, _CH), jnp.int32),
        pltpu.VMEM((2, _CH), jnp.int32),
        pltpu.VMEM((2, _CH, DIM), jnp.float32),
        pltpu.VMEM((2, _CH, DIM), jnp.float32),
        pltpu.VMEM((2, _CH, DIM), jnp.float32),
        pltpu.SemaphoreType.DMA,
        pltpu.SemaphoreType.DMA,
        pltpu.SemaphoreType.DMA,
    ],
)


_BBLK = 512


def _tc_gram_mlp(rp_ref, w1_ref, b1_ref, w2_ref, b2_ref, out_ref):
    rows = [rp_ref[i, :, :] for i in range(NPAIR)]
    ent = {}
    for i in range(NPAIR):
        for j in range(i, NPAIR):
            ent[(i, j)] = jnp.sum(rows[i] * rows[j], axis=1, keepdims=True)
    cols = []
    for i in range(NPAIR):
        for j in range(NPAIR):
            cols.append(ent[(i, j)] if i <= j else ent[(j, i)])
    feat = jnp.concatenate(cols, axis=1)
    feat = jnp.where(feat < 0.0, 0.0, feat)
    feat = jnp.log(feat + 1.0)
    h = jnp.dot(feat, w1_ref[...], preferred_element_type=jnp.float32)
    h = jnp.maximum(h + b1_ref[...], 0.0)
    out_ref[...] = (
        jnp.dot(h, w2_ref[...], preferred_element_type=jnp.float32)
        + b2_ref[...])


def kernel(src_node_ids, dst_node_ids, RP, lambda_weights, W1, b1, W2, b2):
    ids = jnp.concatenate(
        [src_node_ids, dst_node_ids]).astype(jnp.int32)
    rp_flat = RP.reshape(M * K1 * NODE_NUM, DIM)
    lam_flat = lambda_weights.reshape(K1 * M).astype(jnp.float32)

    fused = _sc_gather_call(ids, rp_flat, lam_flat)

    nblk = BATCH // _BBLK
    out = pl.pallas_call(
        _tc_gram_mlp,
        grid=(nblk,),
        in_specs=[
            pl.BlockSpec((NPAIR, _BBLK, DIM), lambda i: (0, i, 0)),
            pl.BlockSpec((PWD, 4 * PWD), lambda i: (0, 0)),
            pl.BlockSpec((1, 4 * PWD), lambda i: (0, 0)),
            pl.BlockSpec((4 * PWD, PWD), lambda i: (0, 0)),
            pl.BlockSpec((1, PWD), lambda i: (0, 0)),
        ],
        out_specs=pl.BlockSpec((_BBLK, PWD), lambda i: (i, 0)),
        out_shape=jax.ShapeDtypeStruct((BATCH, PWD), jnp.float32),
    )(fused, W1, b1.reshape(1, 4 * PWD), W2, b2.reshape(1, PWD))
    return out


# speedup vs baseline: 1.2687x; 1.2687x over previous
import jax
import jax.numpy as jnp
from jax import lax
from jax.experimental import pallas as pl
from jax.experimental.pallas import tpu as pltpu
from jax.experimental.pallas import tpu_sc as plsc

NODE_NUM = 50000
DIM = 128
K1 = 3
M = 2
NPAIR = 2 * K1
PWD = NPAIR * NPAIR
BATCH = 4096

_NC = 2
_NS = 16
_NW = _NC * _NS
_PER_W = (2 * BATCH) // _NW
_CH = 128
_LANES = 16
_NTASK = K1 * (_PER_W // _CH)


def _vgather(x, idx):
    return lax.gather(
        x, idx[:, None],
        lax.GatherDimensionNumbers(offset_dims=(), collapsed_slice_dims=(0,),
                                   start_index_map=(0,)),
        (1,), mode=lax.GatherScatterMode.PROMISE_IN_BOUNDS)


def _sc_fused_gather(ids_hbm, rp_hbm, lam_hbm, out_hbm,
                     idx_v, lam_v, idx0, idx1, b0, b1, fb,
                     gsem_a, gsem_b, osem):
    wid = lax.axis_index("s") * _NC + lax.axis_index("c")
    base = wid * _PER_W
    half = base // BATCH
    brow = base - half * BATCH

    pltpu.sync_copy(ids_hbm.at[pl.ds(base, _PER_W)], idx_v)

    pltpu.sync_copy(lam_hbm, lam_v.at[pl.ds(0, 6)])
    lane = lax.iota(jnp.int32, 16)
    f = lax.rem(lane, 6)
    lv = lam_v[...]
    a = _vgather(lv, f)
    b = _vgather(lv, f ^ 1)
    mx = jnp.maximum(a, b)
    ea = jnp.exp(a - mx)
    eb = jnp.exp(b - mx)
    w = ea / (ea + eb)
    wks = []
    for k in range(K1):
        w0 = _vgather(w, jnp.full((16,), 2 * k, jnp.int32))
        w1 = _vgather(w, jnp.full((16,), 2 * k + 1, jnp.int32))
        wks.append((w0, w1))

    tasks = [(k, c0) for k in range(K1) for c0 in range(0, _PER_W, _CH)]
    gsems = [gsem_a, gsem_b]

    def compute_idx(t, p):
        k, c0 = tasks[t]
        for tt in range(_CH // _LANES):
            sls = pl.ds(c0 + tt * _LANES, _LANES)
            sld = pl.ds(tt * _LANES, _LANES)
            v = idx_v[sls]
            idx0[p, sld] = v + (k * NODE_NUM)
            idx1[p, sld] = v + ((K1 + k) * NODE_NUM)

    def fire(t, p):
        h0 = pltpu.async_copy(rp_hbm.at[idx0.at[p]], b0.at[p], gsems[p])
        h1 = pltpu.async_copy(rp_hbm.at[idx1.at[p]], b1.at[p], gsems[p])
        return h0, h1

    handles = {}
    compute_idx(0, 0)
    handles[0] = fire(0, 0)
    owrite = None
    for t in range(_NTASK):
        p = t % 2
        if t + 1 < _NTASK:
            compute_idx(t + 1, 1 - p)
            handles[t + 1] = fire(t + 1, 1 - p)
        h0, h1 = handles.pop(t)
        h0.wait()
        h1.wait()
        k, c0 = tasks[t]
        w0, w1 = wks[k]

        @plsc.parallel_loop(0, _CH, 1, unroll=2)
        def _fuse_row(c, p=p, w0=w0, w1=w1):
            for l in range(DIM // _LANES):
                sl = pl.ds(l * _LANES, _LANES)
                fb[p, c, sl] = b0[p, c, sl] * w0 + b1[p, c, sl] * w1
        if owrite is not None:
            owrite.wait()
        owrite = pltpu.async_copy(
            fb.at[p], out_hbm.at[half * K1 + k, pl.ds(brow + c0, _CH)], osem)
    owrite.wait()


_sc_gather_call = pl.kernel(
    _sc_fused_gather,
    out_type=jax.ShapeDtypeStruct((NPAIR, BATCH, DIM), jnp.float32),
    mesh=plsc.VectorSubcoreMesh(core_axis_name="c", subcore_axis_name="s"),
    scratch_types=[
        pltpu.VMEM((_PER_W,), jnp.int32),
        pltpu.VMEM((_LANES,), jnp.float32),
        pltpu.VMEM((2, _CH), jnp.int32),
        pltpu.VMEM((2, _CH), jnp.int32),
        pltpu.VMEM((2, _CH, DIM), jnp.float32),
        pltpu.VMEM((2, _CH, DIM), jnp.float32),
        pltpu.VMEM((2, _CH, DIM), jnp.float32),
        pltpu.SemaphoreType.DMA,
        pltpu.SemaphoreType.DMA,
        pltpu.SemaphoreType.DMA,
    ],
)


_BBLK = 512


def _tc_gram_mlp(rp_ref, w1_ref, b1_ref, w2_ref, b2_ref, out_ref):
    rows = [rp_ref[i, :, :] for i in range(NPAIR)]
    ent = {}
    for i in range(NPAIR):
        for j in range(i, NPAIR):
            ent[(i, j)] = jnp.sum(rows[i] * rows[j], axis=1, keepdims=True)
    cols = []
    for i in range(NPAIR):
        for j in range(NPAIR):
            cols.append(ent[(i, j)] if i <= j else ent[(j, i)])
    feat = jnp.concatenate(cols, axis=1)
    feat = jnp.where(feat < 0.0, 0.0, feat)
    feat = jnp.log(feat + 1.0)
    h = jnp.dot(feat, w1_ref[...], preferred_element_type=jnp.float32)
    h = jnp.maximum(h + b1_ref[...], 0.0)
    out_ref[...] = (
        jnp.dot(h, w2_ref[...], preferred_element_type=jnp.float32)
        + b2_ref[...])


def kernel(src_node_ids, dst_node_ids, RP, lambda_weights, W1, b1, W2, b2):
    ids = jnp.concatenate(
        [src_node_ids, dst_node_ids]).astype(jnp.int32)
    rp_flat = RP.reshape(M * K1 * NODE_NUM, DIM)
    lam_flat = lambda_weights.reshape(K1 * M).astype(jnp.float32)

    fused = _sc_gather_call(ids, rp_flat, lam_flat)

    nblk = BATCH // _BBLK
    out = pl.pallas_call(
        _tc_gram_mlp,
        grid=(nblk,),
        in_specs=[
            pl.BlockSpec((NPAIR, _BBLK, DIM), lambda i: (0, i, 0)),
            pl.BlockSpec((PWD, 4 * PWD), lambda i: (0, 0)),
            pl.BlockSpec((1, 4 * PWD), lambda i: (0, 0)),
            pl.BlockSpec((4 * PWD, PWD), lambda i: (0, 0)),
            pl.BlockSpec((1, PWD), lambda i: (0, 0)),
        ],
        out_specs=pl.BlockSpec((_BBLK, PWD), lambda i: (i, 0)),
        out_shape=jax.ShapeDtypeStruct((BATCH, PWD), jnp.float32),
    )(fused, W1, b1.reshape(1, 4 * PWD), W2, b2.reshape(1, PWD))
    return out
